# baseline (speedup 1.0000x reference)
import jax
import jax.numpy as jnp
from jax.experimental import pallas as pl

D2 = 256
M = 1024
N = 128
L = 128
CH = 16

_TN = (((0,), (0,)), ((), ()))
_NT = (((1,), (1,)), ((), ()))
_NN = (((1,), (0,)), ((), ()))


def _body(ctx_ref, query_ref, binM_ref, adjf_ref, adjT_ref, V_ref, U_ref,
          brow_ref, w1row_ref, w2col_ref, Wih_ref, bias_ref, out_ref):
    f32 = jnp.float32
    ctx = ctx_ref[:]
    binM = binM_ref[:]

    mean_pool = jax.lax.dot_general(binM, ctx, _TN,
                                    preferred_element_type=f32) * (1.0 / M)
    q_row = jnp.sum(query_ref[:], axis=0, keepdims=True) * (1.0 / L)
    qV = jax.lax.dot_general(q_row, V_ref[:], _NN,
                             preferred_element_type=f32)
    Wih = Wih_ref[:]
    Wih_x = Wih[:, :D2]

    max_pool = jnp.full((N, D2), -jnp.inf, dtype=f32)
    gx = []
    for i in range(M // CH):
        bchunk = binM[i * CH:(i + 1) * CH, :]
        cchunk = ctx[i * CH:(i + 1) * CH, :]
        vals = bchunk[:, :, None] * cchunk[:, None, :]
        max_pool = jnp.maximum(max_pool, jnp.max(vals, axis=0))
        gx.append(jax.lax.dot_general(cchunk, Wih_x, _NT,
                                      preferred_element_type=f32))
    gates_x = jnp.concatenate(gx, axis=0)

    g_col = (jax.lax.dot_general(mean_pool, qV[:, :D2], _NT,
                                 preferred_element_type=f32)
             + jax.lax.dot_general(max_pool, qV[:, D2:], _NT,
                                   preferred_element_type=f32)) * (1.0 / 16.0)
    gate = jax.nn.sigmoid(g_col)

    U = U_ref[:]
    hidden = gate * (jax.lax.dot_general(mean_pool, U[:, :D2], _NT,
                                         preferred_element_type=f32)
                     + jax.lax.dot_general(max_pool, U[:, D2:], _NT,
                                           preferred_element_type=f32))
    hidden = hidden + brow_ref[:]

    a_row = jax.lax.dot_general(w1row_ref[:], hidden, _NT,
                                preferred_element_type=f32)
    c_col = jax.lax.dot_general(hidden, w2col_ref[:], _NN,
                                preferred_element_type=f32)
    pre = c_col + a_row
    raw_T = jnp.where(pre >= 0.0, pre, 0.01 * pre)
    betas_T = adjT_ref[:] * raw_T
    mx = jnp.max(betas_T, axis=0, keepdims=True)
    e = jnp.exp(betas_T - mx)
    alphas_T = e / jnp.sum(e, axis=0, keepdims=True)
    S = adjf_ref[:] * alphas_T
    E_t = jax.lax.dot_general(S, hidden, _NN,
                              preferred_element_type=f32)
    E_t = jnp.maximum(E_t, 0.0)

    emb_info = jax.lax.dot_general(binM, E_t, _NN,
                                   preferred_element_type=f32)
    gates = (gates_x
             + jax.lax.dot_general(emb_info, Wih[:, D2:], _NT,
                                   preferred_element_type=f32)
             + bias_ref[:])
    i_g = gates[:, :D2]
    g_g = gates[:, D2:2 * D2]
    o_g = gates[:, 2 * D2:]
    c_t = jax.nn.sigmoid(i_g) * jnp.tanh(g_g)
    out_ref[:] = jax.nn.sigmoid(o_g) * jnp.tanh(c_t)


@jax.jit
def _run(context_emb, query_emb, bin_M, adj_f, adjT_f, V, U, b_row,
         w1_row, w2_col, W_ih, bias_row):
    return pl.pallas_call(
        _body,
        out_shape=jax.ShapeDtypeStruct((M, D2), jnp.float32),
    )(context_emb, query_emb, bin_M, adj_f, adjT_f, V, U, b_row,
      w1_row, w2_col, W_ih, bias_row)


def kernel(context_emb, query_emb, bin_M, adj, V, U, b, W, W_ih, W_hh, b_ih, b_hh):
    adj_f = adj.astype(jnp.float32)
    adjT_f = adj_f.T
    b_row = b.reshape(1, D2)
    w1_row = W[:D2, 0].reshape(1, D2)
    w2_col = W[D2:, 0].reshape(D2, 1)
    W_igo = jnp.concatenate([W_ih[:D2], W_ih[2 * D2:]], axis=0)
    bias = b_ih + b_hh
    bias_row = jnp.concatenate([bias[:D2], bias[2 * D2:]]).reshape(1, 3 * D2)
    return _run(context_emb, query_emb, bin_M, adj_f, adjT_f, V, U,
                b_row, w1_row, w2_col, W_igo, bias_row)


# speedup vs baseline: 1.0677x; 1.0677x over previous
import jax
import jax.numpy as jnp
from jax.experimental import pallas as pl

D2 = 256
M = 1024
N = 128
L = 128
CH = 16

_TN = (((0,), (0,)), ((), ()))
_NT = (((1,), (1,)), ((), ()))
_NN = (((1,), (0,)), ((), ()))


def _body(ctx_ref, query_ref, binM_ref, adjf_ref, adjT_ref, V_ref, U_ref,
          brow_ref, w1row_ref, w2col_ref, Wih_ref, bias_ref, out_ref):
    f32 = jnp.float32
    ctx = ctx_ref[:]
    binM = binM_ref[:]

    mean_pool = jax.lax.dot_general(binM, ctx, _TN,
                                    preferred_element_type=f32) * (1.0 / M)
    q_row = jnp.sum(query_ref[:], axis=0, keepdims=True) * (1.0 / L)
    qV = jax.lax.dot_general(q_row, V_ref[:], _NN,
                             preferred_element_type=f32)
    Wih = Wih_ref[:]
    Wih_x = Wih[:, :D2]

    max_pool = jnp.full((N, D2), -jnp.inf, dtype=f32)
    gx = []
    for i in range(M // CH):
        bchunk = binM[i * CH:(i + 1) * CH, :]
        cchunk = ctx[i * CH:(i + 1) * CH, :]
        vals = bchunk[:, :, None] * cchunk[:, None, :]
        max_pool = jnp.maximum(max_pool, jnp.max(vals, axis=0))
        gx.append(jax.lax.dot_general(cchunk, Wih_x, _NT,
                                      preferred_element_type=f32))
    gates_x = jnp.concatenate(gx, axis=0)

    g_col = (jax.lax.dot_general(mean_pool, qV[:, :D2], _NT,
                                 preferred_element_type=f32)
             + jax.lax.dot_general(max_pool, qV[:, D2:], _NT,
                                   preferred_element_type=f32)) * (1.0 / 16.0)
    gate = jax.nn.sigmoid(g_col)

    U = U_ref[:]
    hidden = gate * (jax.lax.dot_general(mean_pool, U[:, :D2], _NT,
                                         preferred_element_type=f32)
                     + jax.lax.dot_general(max_pool, U[:, D2:], _NT,
                                           preferred_element_type=f32))
    hidden = hidden + brow_ref[:]

    a_row = jax.lax.dot_general(w1row_ref[:], hidden, _NT,
                                preferred_element_type=f32)
    c_col = jax.lax.dot_general(hidden, w2col_ref[:], _NN,
                                preferred_element_type=f32)
    pre = c_col + a_row
    raw_T = jnp.where(pre >= 0.0, pre, 0.01 * pre)
    betas_T = adjT_ref[:] * raw_T
    mx = jnp.max(betas_T, axis=0, keepdims=True)
    e = jnp.exp(betas_T - mx)
    alphas_T = e / jnp.sum(e, axis=0, keepdims=True)
    S = adjf_ref[:] * alphas_T
    E_t = jax.lax.dot_general(S, hidden, _NN,
                              preferred_element_type=f32)
    E_t = jnp.maximum(E_t, 0.0)

    emb_info = jax.lax.dot_general(binM, E_t, _NN,
                                   preferred_element_type=f32)
    gates = (gates_x
             + jax.lax.dot_general(emb_info, Wih[:, D2:], _NT,
                                   preferred_element_type=f32)
             + bias_ref[:])
    i_g = gates[:, :D2]
    g_g = gates[:, 2 * D2:3 * D2]
    o_g = gates[:, 3 * D2:]
    c_t = jax.nn.sigmoid(i_g) * jnp.tanh(g_g)
    out_ref[:] = jax.nn.sigmoid(o_g) * jnp.tanh(c_t)


@jax.jit
def _run(context_emb, query_emb, bin_M, adj_f, adjT_f, V, U, b_row,
         w1_row, w2_col, W_ih, bias_row):
    return pl.pallas_call(
        _body,
        out_shape=jax.ShapeDtypeStruct((M, D2), jnp.float32),
    )(context_emb, query_emb, bin_M, adj_f, adjT_f, V, U, b_row,
      w1_row, w2_col, W_ih, bias_row)


def kernel(context_emb, query_emb, bin_M, adj, V, U, b, W, W_ih, W_hh, b_ih, b_hh):
    adj_f = adj.astype(jnp.float32)
    adjT_f = adj_f.T
    b_row = b.reshape(1, D2)
    w1_row = W[:D2, 0].reshape(1, D2)
    w2_col = W[D2:, 0].reshape(D2, 1)
    bias_row = (b_ih + b_hh).reshape(1, 4 * D2)
    return _run(context_emb, query_emb, bin_M, adj_f, adjT_f, V, U,
                b_row, w1_row, w2_col, W_ih, bias_row)
